# baseline (speedup 1.0000x reference)
import jax
import jax.numpy as jnp
from jax import lax
from jax.experimental import pallas as pl
from jax.experimental.pallas import tpu as pltpu

_N, _NP, _F, _H = 4096, 1024, 64, 32
_BM = 512
_NBLK = _N // _BM
_NPBLK = _NP // _BM
_BM2 = 2048
_NBLK2 = _N // _BM2
_F32 = jnp.float32
_BF16 = jnp.bfloat16
_F8 = jnp.float8_e4m3fn
_F8_SCALE = 65536.0


def _dot(a, b):
    return jnp.dot(a, b, preferred_element_type=_F32)


def _fdot(a, b):
    return jnp.dot(a, b, preferred_element_type=_F32,
                   precision=lax.Precision.DEFAULT)


def _bdot(a_f8, b_f32):
    return lax.dot_general(a_f8, b_f32.astype(_BF16),
                           (((1,), (0,)), ((), ())),
                           preferred_element_type=_F32,
                           precision=lax.Precision.DEFAULT)


def _relu(x):
    return jnp.maximum(x, 0.0)


def _callA_body(xn, xp, ap, w1, b1, w14, b14, w2, w25, an,
                anf8_out, p21_out, p22_out, p23_out, p24_out, p25_out,
                pa_ref, p4_ref, pooled_ref):
    i = pl.program_id(0)

    @pl.when(i == 0)
    def _():
        pa_ref[...] = _dot(xn[...], w1[...])
        p4_ref[...] = _dot(xp[...], w14[...])

    @pl.when(i < _NPBLK)
    def _():
        x14 = _relu(_fdot(ap[...], p4_ref[...]) + b14[...])
        ones = jnp.ones((_BM, 1), dtype=_F32)
        part = lax.dot_general(
            x14, ones, (((0,), (0,)), ((), ())), preferred_element_type=_F32
        )
        @pl.when(i == 0)
        def _():
            pooled_ref[...] = jnp.zeros_like(pooled_ref)
        pooled_ref[...] += part

    a_n = an[...]
    anf8_out[...] = (a_n * _F8_SCALE).astype(_F8)
    x1 = _relu(_fdot(a_n, pa_ref[...]) + b1[...])
    x11 = x1[:, 0:_H]
    x12 = x1[:, _H:2 * _H]
    x13 = x1[:, 2 * _H:3 * _H]
    p21_out[...] = _dot(x11, w2[:, 0:_H])
    p22_out[...] = _dot(x12, w2[:, _H:2 * _H])
    p23_out[...] = _dot(x12, w2[:, 2 * _H:3 * _H])
    p24_out[...] = _dot(x13, w2[:, 3 * _H:4 * _H])

    @pl.when(i == _NPBLK)
    def _():
        wsum = jnp.sum(w25[...], axis=0, keepdims=True)
        outer = _dot(pooled_ref[...], wsum)
        r = lax.broadcasted_iota(jnp.int32, (_N, _H), 0) // 128
        c = lax.broadcasted_iota(jnp.int32, (_N, _H), 1)
        p25_out[...] = _dot((r == c).astype(_F32), outer)


_NBUF = 4
_BMB = 256
_NBB = _N // _BMB


def _callB_body(p22, p23, b2, ats_hbm, acs_hbm, sacc_out, tbuf, cbuf, sems):
    def start(k):
        slot = k % _NBUF
        pltpu.make_async_copy(
            ats_hbm.at[pl.ds(k * _BMB, _BMB), :], tbuf.at[slot],
            sems.at[0, slot]).start()
        pltpu.make_async_copy(
            acs_hbm.at[pl.ds(k * _BMB, _BMB), :], cbuf.at[slot],
            sems.at[1, slot]).start()

    for k in range(_NBUF):
        start(k)
    for k in range(_NBB):
        slot = k % _NBUF
        pltpu.make_async_copy(
            ats_hbm.at[pl.ds(k * _BMB, _BMB), :], tbuf.at[slot],
            sems.at[0, slot]).wait()
        pltpu.make_async_copy(
            acs_hbm.at[pl.ds(k * _BMB, _BMB), :], cbuf.at[slot],
            sems.at[1, slot]).wait()
        sacc_out[pl.ds(k * _BMB, _BMB), :] = (
            _relu(_fdot(tbuf[slot], p22[...]) + b2[:, _H:2 * _H])
            + _relu(_fdot(cbuf[slot], p23[...]) + b2[:, 2 * _H:3 * _H]))
        if k + _NBUF < _NBB:
            start(k + _NBUF)


def _callC1_body(p21, p25, p24, sacc, b2, w3, anf8, as_,
                 asf8_out, p3_out):
    a_s = as_[...]
    asf8_out[...] = (a_s * _F8_SCALE).astype(_F8)
    a_bf = anf8[...]
    s = (sacc[...]
         + _relu(_bdot(a_bf, p21[...]) + b2[:, 0:_H])
         + _relu(_bdot(a_bf, p25[...]) + b2[:, 4 * _H:5 * _H])
         + _relu(_fdot(a_s, p24[...]) + b2[:, 3 * _H:4 * _H]))
    p3_out[...] = _dot(s, w3[...])


def _callC2_body(p3, b3, w41, b41, w42, b42, anf8, asf8,
                 o1_out, o2_out, p41_ref, p42_ref):
    p = pl.program_id(0)
    i = pl.program_id(1)
    row = pl.ds(i * _BM2, _BM2)

    @pl.when(p == 0)
    def _():
        t1 = _relu(_bdot(anf8[...], p3[:, 0:_H]) + b3[:, 0:_H])
        t2 = _relu(_bdot(asf8[...], p3[:, _H:2 * _H]) + b3[:, _H:2 * _H])
        p41_ref[row, :] = _dot(t1, w41[...])
        p42_ref[row, :] = _dot(t2, w42[...])

    @pl.when(p == 1)
    def _():
        a_bf = anf8[...]
        o1_out[...] = jax.nn.sigmoid(_bdot(a_bf, p41_ref[...]) + b41[...])
        o2_out[...] = jax.nn.sigmoid(_bdot(a_bf, p42_ref[...]) + b42[...])


def _cparams():
    return pltpu.CompilerParams(
        vmem_limit_bytes=64 * 1024 * 1024,
    )


def kernel(x_n, A_n, A_s, A_n_ts, A_n_cs, x_p, A_p,
           W1_1, b1_1, W1_2, b1_2, W1_3, b1_3, W1_4, b1_4,
           W2_1, b2_1, W2_2, b2_2, W2_3, b2_3, W2_4, b2_4, W2_5, b2_5,
           W3_1, b3_1, W3_2, b3_2, W4_1, b4_1, W4_2, b4_2):
    xn = x_n[0]
    xp = x_p[0]
    an = A_n[0]
    as_ = A_s[0]
    ats = A_n_ts[0]
    acs = A_n_cs[0]
    ap = A_p[0]

    w1 = jnp.concatenate([W1_1, W1_2, W1_3], axis=1)
    b1 = jnp.concatenate([b1_1, b1_2, b1_3])[None, :]
    inv = 1.0 / _F8_SCALE
    w2 = jnp.concatenate([W2_1 * inv, W2_2, W2_3, W2_4], axis=1)
    b2 = jnp.concatenate([b2_1, b2_2, b2_3, b2_4, b2_5])[None, :]
    w3 = jnp.concatenate([W3_1 * inv, W3_2 * inv], axis=1)
    b3 = jnp.concatenate([b3_1, b3_2])[None, :]

    a1 = W4_1.shape[1]
    a2 = W4_2.shape[1]

    def full(shape):
        return pl.BlockSpec(shape, lambda *idx: (0,) * len(shape))

    def rows1(width):
        return pl.BlockSpec((_BM, width), lambda i: (i, 0))

    anf8, p21, p22, p23, p24, p25 = pl.pallas_call(
        _callA_body,
        grid=(_NBLK,),
        in_specs=[
            full((_N, _F)), full((_NP, _F)),
            pl.BlockSpec((_BM, _NP), lambda i: (jnp.minimum(i, _NPBLK - 1), 0)),
            full((_F, 3 * _H)), full((1, 3 * _H)),
            full((_F, _H)), full((1, _H)),
            full((_H, 4 * _H)), full((_H, _H)),
            rows1(_N),
        ],
        out_specs=[
            rows1(_N), rows1(_H), rows1(_H), rows1(_H), rows1(_H),
            full((_N, _H)),
        ],
        out_shape=[
            jax.ShapeDtypeStruct((_N, _N), _F8),
            jax.ShapeDtypeStruct((_N, _H), _F32),
            jax.ShapeDtypeStruct((_N, _H), _F32),
            jax.ShapeDtypeStruct((_N, _H), _F32),
            jax.ShapeDtypeStruct((_N, _H), _F32),
            jax.ShapeDtypeStruct((_N, _H), _F32),
        ],
        scratch_shapes=[
            pltpu.VMEM((_N, 3 * _H), _F32),
            pltpu.VMEM((_NP, _H), _F32),
            pltpu.VMEM((_H, 1), _F32),
        ],
        compiler_params=_cparams(),
    )(xn, xp, ap, w1, b1, W1_4, b1_4[None, :], w2, W2_5 * inv, an)

    sacc = pl.pallas_call(
        _callB_body,
        in_specs=[
            full((_N, _H)), full((_N, _H)), full((1, 5 * _H)),
            pl.BlockSpec(memory_space=pl.ANY),
            pl.BlockSpec(memory_space=pl.ANY),
        ],
        out_specs=pl.BlockSpec((_N, _H), lambda: (0, 0)),
        out_shape=jax.ShapeDtypeStruct((_N, _H), _F32),
        scratch_shapes=[
            pltpu.VMEM((_NBUF, _BMB, _N), _F32),
            pltpu.VMEM((_NBUF, _BMB, _N), _F32),
            pltpu.SemaphoreType.DMA((2, _NBUF)),
        ],
        compiler_params=_cparams(),
    )(p22, p23, b2, ats, acs)

    asf8, p3 = pl.pallas_call(
        _callC1_body,
        grid=(_NBLK,),
        in_specs=[
            full((_N, _H)), full((_N, _H)), full((_N, _H)),
            rows1(_H), full((1, 5 * _H)), full((_H, 2 * _H)),
            rows1(_N), rows1(_N),
        ],
        out_specs=[rows1(_N), rows1(2 * _H)],
        out_shape=[
            jax.ShapeDtypeStruct((_N, _N), _F8),
            jax.ShapeDtypeStruct((_N, 2 * _H), _F32),
        ],
        compiler_params=_cparams(),
    )(p21, p25, p24, sacc, b2, w3, anf8, as_)

    out1, out2 = pl.pallas_call(
        _callC2_body,
        grid=(2, _NBLK2),
        in_specs=[
            full((_N, 2 * _H)), full((1, 2 * _H)),
            full((_H, a1)), full((1, a1)),
            full((_H, a2)), full((1, a2)),
            pl.BlockSpec((_BM2, _N), lambda p, i: (i, 0)),
            pl.BlockSpec((_BM2, _N),
                         lambda p, i: (jnp.where(p == 0, i, _NBLK2 - 1), 0)),
        ],
        out_specs=[
            pl.BlockSpec((_BM2, a1), lambda p, i: (jnp.where(p == 1, i, 0), 0)),
            pl.BlockSpec((_BM2, a2), lambda p, i: (jnp.where(p == 1, i, 0), 0)),
        ],
        out_shape=[
            jax.ShapeDtypeStruct((_N, a1), _F32),
            jax.ShapeDtypeStruct((_N, a2), _F32),
        ],
        scratch_shapes=[
            pltpu.VMEM((_N, a1), _F32),
            pltpu.VMEM((_N, a2), _F32),
        ],
        compiler_params=_cparams(),
    )(p3, b3, W4_1 * inv, b4_1[None, :], W4_2 * inv, b4_2[None, :],
      anf8, asf8)

    return (out1[None], out2[None])


# speedup vs baseline: 1.0623x; 1.0623x over previous
import jax
import jax.numpy as jnp
from jax import lax
from jax.experimental import pallas as pl
from jax.experimental.pallas import tpu as pltpu

_N, _NP, _F, _H = 4096, 1024, 64, 32
_BM = 512
_NBLK = _N // _BM
_NPBLK = _NP // _BM
_F32 = jnp.float32
_BF16 = jnp.bfloat16
_F8 = jnp.float8_e4m3fn
_F8_SCALE = 65536.0


def _dot(a, b):
    return jnp.dot(a, b, preferred_element_type=_F32)


def _fdot(a, b):
    return jnp.dot(a, b, preferred_element_type=_F32,
                   precision=lax.Precision.DEFAULT)


def _bdot(a_f8, b_f32):
    return lax.dot_general(a_f8, b_f32.astype(_BF16),
                           (((1,), (0,)), ((), ())),
                           preferred_element_type=_F32,
                           precision=lax.Precision.DEFAULT)


def _relu(x):
    return jnp.maximum(x, 0.0)


def _callA_body(xn, xp, ap, w1, b1, w14, b14, w2, w25, an,
                anf8_out, p21_out, p22_out, p23_out, p24_out, p25_out,
                pa_ref, p4_ref, pooled_ref):
    i = pl.program_id(0)

    @pl.when(i == 0)
    def _():
        pa_ref[...] = _dot(xn[...], w1[...])
        p4_ref[...] = _dot(xp[...], w14[...])

    @pl.when(i < _NPBLK)
    def _():
        x14 = _relu(_fdot(ap[...], p4_ref[...]) + b14[...])
        ones = jnp.ones((_BM, 1), dtype=_F32)
        part = lax.dot_general(
            x14, ones, (((0,), (0,)), ((), ())), preferred_element_type=_F32
        )
        @pl.when(i == 0)
        def _():
            pooled_ref[...] = jnp.zeros_like(pooled_ref)
        pooled_ref[...] += part

    a_n = an[...]
    anf8_out[...] = (a_n * _F8_SCALE).astype(_F8)
    x1 = _relu(_fdot(a_n, pa_ref[...]) + b1[...])
    x11 = x1[:, 0:_H]
    x12 = x1[:, _H:2 * _H]
    x13 = x1[:, 2 * _H:3 * _H]
    p21_out[...] = _dot(x11, w2[:, 0:_H])
    p22_out[...] = _dot(x12, w2[:, _H:2 * _H])
    p23_out[...] = _dot(x12, w2[:, 2 * _H:3 * _H])
    p24_out[...] = _dot(x13, w2[:, 3 * _H:4 * _H])

    @pl.when(i == _NPBLK)
    def _():
        wsum = jnp.sum(w25[...], axis=0, keepdims=True)
        outer = _dot(pooled_ref[...], wsum)
        r = lax.broadcasted_iota(jnp.int32, (_N, _H), 0) // 128
        c = lax.broadcasted_iota(jnp.int32, (_N, _H), 1)
        p25_out[...] = _dot((r == c).astype(_F32), outer)


def _callB_body(p22, p23, b2, ats, acs, sacc_out):
    sacc_out[...] = (
        _relu(_fdot(ats[...], p22[...]) + b2[:, _H:2 * _H])
        + _relu(_fdot(acs[...], p23[...]) + b2[:, 2 * _H:3 * _H]))


def _callC_body(p21, p25, p24, sacc, b2, w3, b3, w41, b41, w42, b42,
                anf8, as_, o1_out, o2_out, asq_ref, p3_ref, p41_ref, p42_ref):
    p = pl.program_id(0)
    i = pl.program_id(1)
    row = pl.ds(i * _BM, _BM)
    a1 = w41.shape[1]
    a2 = w42.shape[1]

    @pl.when(p == 0)
    def _():
        a_s = as_[...]
        asq_ref[row, :] = (a_s * _F8_SCALE).astype(_F8)
        a_f8 = anf8[...]
        s = (sacc[...]
             + _relu(_bdot(a_f8, p21[...]) + b2[:, 0:_H])
             + _relu(_bdot(a_f8, p25[...]) + b2[:, 4 * _H:5 * _H])
             + _relu(_fdot(a_s, p24[...]) + b2[:, 3 * _H:4 * _H]))
        p3_ref[row, :] = _dot(s, w3[...])

    @pl.when(p == 1)
    def _():
        t1 = _relu(_bdot(anf8[...], p3_ref[:, 0:_H]) + b3[:, 0:_H])
        t2 = _relu(_bdot(asq_ref[row, :], p3_ref[:, _H:2 * _H])
                   + b3[:, _H:2 * _H])
        p41_ref[row, :] = _dot(t1, w41[...])
        p42_ref[row, :] = _dot(t2, w42[...])

    @pl.when(p == 2)
    def _():
        a_f8 = anf8[...]
        o1_out[...] = jax.nn.sigmoid(_bdot(a_f8, p41_ref[...]) + b41[...])
        o2_out[...] = jax.nn.sigmoid(_bdot(a_f8, p42_ref[...]) + b42[...])


def _cparams():
    return pltpu.CompilerParams(
        vmem_limit_bytes=64 * 1024 * 1024,
    )


def kernel(x_n, A_n, A_s, A_n_ts, A_n_cs, x_p, A_p,
           W1_1, b1_1, W1_2, b1_2, W1_3, b1_3, W1_4, b1_4,
           W2_1, b2_1, W2_2, b2_2, W2_3, b2_3, W2_4, b2_4, W2_5, b2_5,
           W3_1, b3_1, W3_2, b3_2, W4_1, b4_1, W4_2, b4_2):
    xn = x_n[0]
    xp = x_p[0]
    an = A_n[0]
    as_ = A_s[0]
    ats = A_n_ts[0]
    acs = A_n_cs[0]
    ap = A_p[0]

    w1 = jnp.concatenate([W1_1, W1_2, W1_3], axis=1)
    b1 = jnp.concatenate([b1_1, b1_2, b1_3])[None, :]
    inv = 1.0 / _F8_SCALE
    w2 = jnp.concatenate([W2_1 * inv, W2_2, W2_3, W2_4], axis=1)
    b2 = jnp.concatenate([b2_1, b2_2, b2_3, b2_4, b2_5])[None, :]
    w3 = jnp.concatenate([W3_1 * inv, W3_2 * inv], axis=1)
    b3 = jnp.concatenate([b3_1, b3_2])[None, :]

    a1 = W4_1.shape[1]
    a2 = W4_2.shape[1]

    def full(shape):
        return pl.BlockSpec(shape, lambda *idx: (0,) * len(shape))

    def rows1(width):
        return pl.BlockSpec((_BM, width), lambda i: (i, 0))

    anf8, p21, p22, p23, p24, p25 = pl.pallas_call(
        _callA_body,
        grid=(_NBLK,),
        in_specs=[
            full((_N, _F)), full((_NP, _F)),
            pl.BlockSpec((_BM, _NP), lambda i: (jnp.minimum(i, _NPBLK - 1), 0)),
            full((_F, 3 * _H)), full((1, 3 * _H)),
            full((_F, _H)), full((1, _H)),
            full((_H, 4 * _H)), full((_H, _H)),
            rows1(_N),
        ],
        out_specs=[
            rows1(_N), rows1(_H), rows1(_H), rows1(_H), rows1(_H),
            full((_N, _H)),
        ],
        out_shape=[
            jax.ShapeDtypeStruct((_N, _N), _F8),
            jax.ShapeDtypeStruct((_N, _H), _F32),
            jax.ShapeDtypeStruct((_N, _H), _F32),
            jax.ShapeDtypeStruct((_N, _H), _F32),
            jax.ShapeDtypeStruct((_N, _H), _F32),
            jax.ShapeDtypeStruct((_N, _H), _F32),
        ],
        scratch_shapes=[
            pltpu.VMEM((_N, 3 * _H), _F32),
            pltpu.VMEM((_NP, _H), _F32),
            pltpu.VMEM((_H, 1), _F32),
        ],
        compiler_params=_cparams(),
    )(xn, xp, ap, w1, b1, W1_4, b1_4[None, :], w2, W2_5 * inv, an)

    sacc = pl.pallas_call(
        _callB_body,
        grid=(_NBLK,),
        in_specs=[
            full((_N, _H)), full((_N, _H)), full((1, 5 * _H)),
            rows1(_N), rows1(_N),
        ],
        out_specs=rows1(_H),
        out_shape=jax.ShapeDtypeStruct((_N, _H), _F32),
        compiler_params=_cparams(),
    )(p22, p23, b2, ats, acs)

    out1, out2 = pl.pallas_call(
        _callC_body,
        grid=(3, _NBLK),
        in_specs=[
            full((_N, _H)), full((_N, _H)), full((_N, _H)),
            pl.BlockSpec((_BM, _H), lambda p, i: (jnp.where(p == 0, i, 0), 0)),
            full((1, 5 * _H)),
            full((_H, 2 * _H)), full((1, 2 * _H)),
            full((_H, a1)), full((1, a1)),
            full((_H, a2)), full((1, a2)),
            pl.BlockSpec((_BM, _N), lambda p, i: (i, 0)),
            pl.BlockSpec((_BM, _N),
                         lambda p, i: (jnp.where(p == 0, i, _NBLK - 1), 0)),
        ],
        out_specs=[
            pl.BlockSpec((_BM, a1), lambda p, i: (jnp.where(p == 2, i, 0), 0)),
            pl.BlockSpec((_BM, a2), lambda p, i: (jnp.where(p == 2, i, 0), 0)),
        ],
        out_shape=[
            jax.ShapeDtypeStruct((_N, a1), _F32),
            jax.ShapeDtypeStruct((_N, a2), _F32),
        ],
        scratch_shapes=[
            pltpu.VMEM((_N, _N), _F8),
            pltpu.VMEM((_N, 2 * _H), _F32),
            pltpu.VMEM((_N, a1), _F32),
            pltpu.VMEM((_N, a2), _F32),
        ],
        compiler_params=_cparams(),
    )(p21, p25, p24, sacc, b2, w3, b3,
      W4_1 * inv, b4_1[None, :], W4_2 * inv, b4_2[None, :], anf8, as_)

    return (out1[None], out2[None])
